# Initial kernel scaffold; baseline (speedup 1.0000x reference)
import jax
import jax.numpy as jnp
from jax.experimental import pallas as pl


def kernel(x, edge_index, edge_attr, batch, params):
    raise NotImplementedError("write your pallas kernel here")


# speedup vs baseline: 5.5189x; 5.5189x over previous
import functools

import jax
import jax.numpy as jnp
from jax.experimental import pallas as pl

HEADS = 4
EMB = 64
HD = HEADS * EMB
N_GRAPHS = 128
RATIO = 0.5


def _row_tile(m):
    for t in (3200, 2000, 1000, 400, 128, 8):
        if m % t == 0:
            return t
    return m


def _mm_body(nargs, x_ref, w_ref, *rest):
    o_ref = rest[-1]
    refs = rest[:-1]
    y = jnp.dot(x_ref[...], w_ref[...], preferred_element_type=jnp.float32)
    i = 0
    b = refs[i][...] if nargs["bias"] else None
    if nargs["bias"]:
        y = y + b
        i += 1
    if nargs["relu"]:
        y = jnp.maximum(y, 0.0)
    if nargs["bn"]:
        y = y * refs[i][...] + refs[i + 1][...]
        i += 2
    if nargs["mask"]:
        y = y * refs[i][...]
        i += 1
    o_ref[...] = y


def _matmul(x, W, b=None, relu=False, bn=None, mask=None):
    m, k = x.shape
    n = W.shape[1]
    tm = _row_tile(m)
    grid = (m // tm,)
    in_specs = [
        pl.BlockSpec((tm, k), lambda i: (i, 0)),
        pl.BlockSpec((k, n), lambda i: (0, 0)),
    ]
    args = [x, W]
    nargs = {"bias": b is not None, "relu": relu, "bn": bn is not None,
             "mask": mask is not None}
    if b is not None:
        args.append(b.reshape(1, n))
        in_specs.append(pl.BlockSpec((1, n), lambda i: (0, 0)))
    if bn is not None:
        scale, shift = bn
        args += [scale.reshape(1, n), shift.reshape(1, n)]
        in_specs += [pl.BlockSpec((1, n), lambda i: (0, 0)),
                     pl.BlockSpec((1, n), lambda i: (0, 0))]
    if mask is not None:
        args.append(mask.reshape(m, 1))
        in_specs.append(pl.BlockSpec((tm, 1), lambda i: (i, 0)))
    return pl.pallas_call(
        functools.partial(_mm_body, nargs),
        grid=grid,
        in_specs=in_specs,
        out_specs=pl.BlockSpec((tm, n), lambda i: (i, 0)),
        out_shape=jax.ShapeDtypeStruct((m, n), jnp.float32),
    )(*args)


def _alpha_body(qd_ref, ks_ref, ep_ref, ekf_ref, o_ref):
    s = qd_ref[...] * (ks_ref[...] + ep_ref[...])
    cols = [s[:, h * EMB:(h + 1) * EMB].sum(axis=1, keepdims=True)
            for h in range(HEADS)]
    alpha = jnp.concatenate(cols, axis=1) / jnp.sqrt(jnp.float32(EMB))
    o_ref[...] = jnp.where(ekf_ref[...] > 0.0, alpha, -1e9)


def _edge_alpha(qd, ks, ep, ekf):
    e = qd.shape[0]
    te = _row_tile(e)
    return pl.pallas_call(
        _alpha_body,
        grid=(e // te,),
        in_specs=[
            pl.BlockSpec((te, HD), lambda i: (i, 0)),
            pl.BlockSpec((te, HD), lambda i: (i, 0)),
            pl.BlockSpec((te, HD), lambda i: (i, 0)),
            pl.BlockSpec((te, 1), lambda i: (i, 0)),
        ],
        out_specs=pl.BlockSpec((te, HEADS), lambda i: (i, 0)),
        out_shape=jax.ShapeDtypeStruct((e, HEADS), jnp.float32),
    )(qd, ks, ep, ekf)


def _w_body(alpha_ref, amaxd_ref, ekf_ref, o_ref):
    o_ref[...] = jnp.exp(alpha_ref[...] - amaxd_ref[...]) * ekf_ref[...]


def _edge_w(alpha, amaxd, ekf):
    e = alpha.shape[0]
    te = _row_tile(e)
    return pl.pallas_call(
        _w_body,
        grid=(e // te,),
        in_specs=[
            pl.BlockSpec((te, HEADS), lambda i: (i, 0)),
            pl.BlockSpec((te, HEADS), lambda i: (i, 0)),
            pl.BlockSpec((te, 1), lambda i: (i, 0)),
        ],
        out_specs=pl.BlockSpec((te, HEADS), lambda i: (i, 0)),
        out_shape=jax.ShapeDtypeStruct((e, HEADS), jnp.float32),
    )(alpha, amaxd, ekf)


def _msg_body(vs_ref, ep_ref, w_ref, dend_ref, o_ref):
    v = vs_ref[...] + ep_ref[...]
    wn = w_ref[...] / jnp.maximum(dend_ref[...], 1e-16)
    parts = [v[:, h * EMB:(h + 1) * EMB] * wn[:, h:h + 1] for h in range(HEADS)]
    o_ref[...] = jnp.concatenate(parts, axis=1)


def _edge_msg(vs, ep, w, dend):
    e = vs.shape[0]
    te = _row_tile(e)
    return pl.pallas_call(
        _msg_body,
        grid=(e // te,),
        in_specs=[
            pl.BlockSpec((te, HD), lambda i: (i, 0)),
            pl.BlockSpec((te, HD), lambda i: (i, 0)),
            pl.BlockSpec((te, HEADS), lambda i: (i, 0)),
            pl.BlockSpec((te, HEADS), lambda i: (i, 0)),
        ],
        out_specs=pl.BlockSpec((te, HD), lambda i: (i, 0)),
        out_shape=jax.ShapeDtypeStruct((e, HD), jnp.float32),
    )(vs, ep, w, dend)


def _gate_body(out_ref, xr_ref, wb_ref, o_ref):
    out = out_ref[...]
    xr = xr_ref[...]
    wb = wb_ref[...]
    logits = (jnp.dot(out, wb[0:HD], preferred_element_type=jnp.float32)
              + jnp.dot(xr, wb[HD:2 * HD], preferred_element_type=jnp.float32)
              + jnp.dot(out - xr, wb[2 * HD:3 * HD],
                        preferred_element_type=jnp.float32))
    beta = jax.nn.sigmoid(logits)
    o_ref[...] = beta * xr + (1.0 - beta) * out


def _gate(out, xr, wbeta):
    m = out.shape[0]
    tm = _row_tile(m)
    return pl.pallas_call(
        _gate_body,
        grid=(m // tm,),
        in_specs=[
            pl.BlockSpec((tm, HD), lambda i: (i, 0)),
            pl.BlockSpec((tm, HD), lambda i: (i, 0)),
            pl.BlockSpec((3 * HD, 1), lambda i: (0, 0)),
        ],
        out_specs=pl.BlockSpec((tm, HD), lambda i: (i, 0)),
        out_shape=jax.ShapeDtypeStruct((m, HD), jnp.float32),
    )(out, xr, wbeta)


def _head_body(h_ref, w1_ref, b1_ref, w2_ref, b2_ref, w3_ref, b3_ref, o_ref):
    a = jnp.maximum(jnp.dot(h_ref[...], w1_ref[...],
                            preferred_element_type=jnp.float32) + b1_ref[...], 0.0)
    a = jnp.maximum(jnp.dot(a, w2_ref[...],
                            preferred_element_type=jnp.float32) + b2_ref[...], 0.0)
    o_ref[...] = jnp.dot(a, w3_ref[...],
                         preferred_element_type=jnp.float32) + b3_ref[...]


def _head(h, p1, p2, p3):
    g = h.shape[0]
    return pl.pallas_call(
        _head_body,
        out_shape=jax.ShapeDtypeStruct((g, 1), jnp.float32),
    )(h, p1["W"], p1["b"].reshape(1, -1), p2["W"], p2["b"].reshape(1, -1),
      p3["W"], p3["b"].reshape(1, -1))


def _transformer_conv(x, src, dst, edge_attr, ekf, p):
    n = x.shape[0]
    q = _matmul(x, p["q"]["W"], p["q"]["b"])
    k = _matmul(x, p["k"]["W"], p["k"]["b"])
    v = _matmul(x, p["v"]["W"], p["v"]["b"])
    xr = _matmul(x, p["s"]["W"], p["s"]["b"])
    ep = _matmul(edge_attr, p["e"]["W"])
    qd = q[dst]
    ks = k[src]
    vs = v[src]
    alpha = _edge_alpha(qd, ks, ep, ekf)
    amax = jax.ops.segment_max(alpha, dst, num_segments=n)
    amax = jnp.where(jnp.isfinite(amax), amax, 0.0)
    w = _edge_w(alpha, amax[dst], ekf)
    denom = jax.ops.segment_sum(w, dst, num_segments=n)
    msg = _edge_msg(vs, ep, w, denom[dst])
    out = jax.ops.segment_sum(msg, dst, num_segments=n)
    return _gate(out, xr, p["beta"]["W"])


def _topk_pool(x, batch, node_keep, weight, ratio, n_graphs):
    n = x.shape[0]
    score = (x @ weight) / (jnp.linalg.norm(weight) + 1e-16)
    score_sel = jnp.where(node_keep, score, -1e9)
    order = jnp.lexsort((-score_sel, batch))
    counts_all = jnp.bincount(batch, length=n_graphs)
    start = jnp.concatenate([jnp.zeros((1,), counts_all.dtype),
                             jnp.cumsum(counts_all)[:-1]])
    pos = jnp.arange(n) - start[batch[order]]
    rank = jnp.zeros((n,), pos.dtype).at[order].set(pos)
    kept = jax.ops.segment_sum(node_keep.astype(jnp.float32), batch,
                               num_segments=n_graphs)
    k_g = jnp.ceil(ratio * kept)
    keep_new = (rank < k_g[batch]) & node_keep
    x_new = x * jnp.tanh(score)[:, None] * keep_new[:, None].astype(x.dtype)
    return x_new, keep_new


def _readout(x, batch, node_keep, n_graphs):
    cnt = jax.ops.segment_sum(node_keep.astype(x.dtype), batch,
                              num_segments=n_graphs)
    neg = jnp.where(node_keep[:, None], x, -1e30)
    gmax = jax.ops.segment_max(neg, batch, num_segments=n_graphs)
    gmax = jnp.where(cnt[:, None] > 0, gmax, 0.0)
    gsum = jax.ops.segment_sum(x * node_keep[:, None].astype(x.dtype), batch,
                               num_segments=n_graphs)
    gmean = gsum / jnp.maximum(cnt, 1.0)[:, None]
    return jnp.concatenate([gmax, gmean], axis=1)


def _bn_pair(p, eps=1e-5):
    return (p["g"] / jnp.sqrt(1.0 + eps), p["b"])


def kernel(x, edge_index, edge_attr, batch, params):
    n = x.shape[0]
    src = edge_index[0]
    dst = edge_index[1]
    node_keep = jnp.ones((n,), bool)
    ekf = jnp.ones((src.shape[0], 1), jnp.float32)

    x = _transformer_conv(x, src, dst, edge_attr, ekf, params["conv1"])
    x = _matmul(x, params["transf1"]["W"], params["transf1"]["b"], relu=True,
                bn=_bn_pair(params["bn1"]))
    reps = []
    for i in range(4):
        x = _transformer_conv(x, src, dst, edge_attr, ekf, params["convs"][i])
        x = _matmul(x, params["transfs"][i]["W"], params["transfs"][i]["b"],
                    relu=True, bn=_bn_pair(params["bns"][i]),
                    mask=node_keep.astype(jnp.float32))
        if i % 2 == 0:
            x, node_keep = _topk_pool(x, batch, node_keep,
                                      params["pools"][i // 2], RATIO, N_GRAPHS)
            ekf = (ekf[:, 0] * node_keep[src].astype(jnp.float32)
                   * node_keep[dst].astype(jnp.float32))[:, None]
            reps.append(_readout(x, batch, node_keep, N_GRAPHS))
    h = reps[0] + reps[1]
    return _head(h, params["lin1"], params["lin2"], params["lin3"])
